# baseline (speedup 1.0000x reference)
import jax
import jax.numpy as jnp
from jax.experimental import pallas as pl
from jax.experimental.pallas import tpu as pltpu

T, N, D, H = 512, 64, 1024, 1024
TB = 16
CH = 4


def _sig(v):
    return 1.0 / (1.0 + jnp.exp(-v))


def _tanh(v):
    return 2.0 / (1.0 + jnp.exp(-2.0 * v)) - 1.0


def _fused_kernel(x_ref, mask_ref, hx_ref, wih_ref, whh_ref, bih_ref,
                  bhh_ref, out_ref, hfin_ref, h_scr, gi_scr):
    k = pl.program_id(0)

    @pl.when(k == 0)
    def _init():
        h_scr[...] = hx_ref[...]

    def gi_chunk(c):
        lo = c * CH * N
        hi = (c + 1) * CH * N
        gi_scr[lo:hi, :] = (
            jnp.dot(x_ref[lo:hi, :], wih_ref[...],
                    preferred_element_type=jnp.float32) + bih_ref[...])

    gi_chunk(0)
    h = h_scr[...]
    for j in range(TB):
        if j % CH == 0 and j // CH + 1 < TB // CH:
            gi_chunk(j // CH + 1)
        gi = gi_scr[j * N:(j + 1) * N, :]
        m = mask_ref[j][:, 0:1]
        h = h * m
        gh = (jnp.dot(h, whh_ref[...], preferred_element_type=jnp.float32)
              + bhh_ref[...])
        r = _sig(gi[:, :H] + gh[:, :H])
        z = _sig(gi[:, H:2 * H] + gh[:, H:2 * H])
        n = _tanh(gi[:, 2 * H:] + r * gh[:, 2 * H:])
        h = (1.0 - z) * n + z * h
        out_ref[j] = h
    h_scr[...] = h

    @pl.when(k == T // TB - 1)
    def _fin():
        hfin_ref[...] = h


def kernel(x, hx, mask, w_ih, w_hh, b_ih, b_hh):
    x2 = x.reshape(T * N, D)
    w_ihT = w_ih.T
    w_hhT = w_hh.T
    b_ih2 = b_ih.reshape(1, 3 * H)
    b_hh2 = b_hh.reshape(1, 3 * H)
    maskB = jnp.broadcast_to(mask[:, :, None], (T, N, 128))

    out, h_final = pl.pallas_call(
        _fused_kernel,
        grid=(T // TB,),
        in_specs=[
            pl.BlockSpec((TB * N, D), lambda k: (k, 0)),
            pl.BlockSpec((TB, N, 128), lambda k: (k, 0, 0)),
            pl.BlockSpec((N, H), lambda k: (0, 0)),
            pl.BlockSpec((D, 3 * H), lambda k: (0, 0)),
            pl.BlockSpec((H, 3 * H), lambda k: (0, 0)),
            pl.BlockSpec((1, 3 * H), lambda k: (0, 0)),
            pl.BlockSpec((1, 3 * H), lambda k: (0, 0)),
        ],
        out_specs=[
            pl.BlockSpec((TB, N, H), lambda k: (k, 0, 0)),
            pl.BlockSpec((N, H), lambda k: (0, 0)),
        ],
        out_shape=[
            jax.ShapeDtypeStruct((T, N, H), jnp.float32),
            jax.ShapeDtypeStruct((N, H), jnp.float32),
        ],
        scratch_shapes=[
            pltpu.VMEM((N, H), jnp.float32),
            pltpu.VMEM((TB * N, 3 * H), jnp.float32),
        ],
        compiler_params=pltpu.CompilerParams(
            dimension_semantics=("arbitrary",),
        ),
    )(x2, maskB, hx, w_ihT, w_hhT, b_ih2, b_hh2)
    return out, h_final


# speedup vs baseline: 1.0236x; 1.0236x over previous
import jax
import jax.numpy as jnp
from jax.experimental import pallas as pl
from jax.experimental.pallas import tpu as pltpu

T, N, D, H = 512, 64, 1024, 1024
TB = 16
CH = 2


def _fused_kernel(x_ref, mask_ref, hx_ref, wih_ref, whh_ref, bih_ref,
                  bhh_ref, out_ref, hfin_ref, h_scr, gi_scr):
    k = pl.program_id(0)

    @pl.when(k == 0)
    def _init():
        h_scr[...] = hx_ref[...]

    def gi_chunk(c):
        lo = c * CH * N
        hi = (c + 1) * CH * N
        gi_scr[lo:hi, :] = (
            jnp.dot(x_ref[lo:hi, :], wih_ref[...],
                    preferred_element_type=jnp.float32) + bih_ref[...])

    gi_chunk(0)
    h = h_scr[...]
    for j in range(TB):
        if j % CH == 0 and j // CH + 1 < TB // CH:
            gi_chunk(j // CH + 1)
        gi = gi_scr[j * N:(j + 1) * N, :]
        m = mask_ref[j][:, 0:1]
        h = h * m
        gh = (jnp.dot(h, whh_ref[...], preferred_element_type=jnp.float32)
              + bhh_ref[...])
        r = jax.nn.sigmoid(gi[:, :H] + gh[:, :H])
        z = jax.nn.sigmoid(gi[:, H:2 * H] + gh[:, H:2 * H])
        n = jnp.tanh(gi[:, 2 * H:] + r * gh[:, 2 * H:])
        h = (1.0 - z) * n + z * h
        out_ref[j] = h
    h_scr[...] = h

    @pl.when(k == T // TB - 1)
    def _fin():
        hfin_ref[...] = h


def kernel(x, hx, mask, w_ih, w_hh, b_ih, b_hh):
    x2 = x.reshape(T * N, D)
    w_ihT = w_ih.T
    w_hhT = w_hh.T
    b_ih2 = b_ih.reshape(1, 3 * H)
    b_hh2 = b_hh.reshape(1, 3 * H)
    maskB = jnp.broadcast_to(mask[:, :, None], (T, N, 128))

    out, h_final = pl.pallas_call(
        _fused_kernel,
        grid=(T // TB,),
        in_specs=[
            pl.BlockSpec((TB * N, D), lambda k: (k, 0)),
            pl.BlockSpec((TB, N, 128), lambda k: (k, 0, 0)),
            pl.BlockSpec((N, H), lambda k: (0, 0)),
            pl.BlockSpec((D, 3 * H), lambda k: (0, 0)),
            pl.BlockSpec((H, 3 * H), lambda k: (0, 0)),
            pl.BlockSpec((1, 3 * H), lambda k: (0, 0)),
            pl.BlockSpec((1, 3 * H), lambda k: (0, 0)),
        ],
        out_specs=[
            pl.BlockSpec((TB, N, H), lambda k: (k, 0, 0)),
            pl.BlockSpec((N, H), lambda k: (0, 0)),
        ],
        out_shape=[
            jax.ShapeDtypeStruct((T, N, H), jnp.float32),
            jax.ShapeDtypeStruct((N, H), jnp.float32),
        ],
        scratch_shapes=[
            pltpu.VMEM((N, H), jnp.float32),
            pltpu.VMEM((TB * N, 3 * H), jnp.float32),
        ],
        compiler_params=pltpu.CompilerParams(
            dimension_semantics=("arbitrary",),
        ),
    )(x2, maskB, hx, w_ihT, w_hhT, b_ih2, b_hh2)
    return out, h_final
